# baseline (speedup 1.0000x reference)
import functools

import jax
import jax.numpy as jnp
from jax import lax
from jax.experimental import pallas as pl
from jax.experimental.pallas import tpu as pltpu
from jax.experimental.pallas import tpu_sc as plsc

N = 50000
E = 800000
EF = 40
L = 3
NF = 64

NP = 51200
HALF = 25600
NHB = 25
EP = 802816
GW = 128
BE = 4096
BN = 1024
SP_ROWS = 25664
TRASH = 25600
SENTINEL = 60000

_GAMMA = (EF - 1) / 8.0


def _emb_kernel(af_ref, w_ref, b_ref, h_ref):
    h_ref[...] = (
        jnp.dot(af_ref[...], w_ref[...], preferred_element_type=jnp.float32)
        + b_ref[...]
    )


def _proj_kernel(h_ref, ws_ref, wd_ref, ts_ref, td_ref):
    h = h_ref[...]
    ts_ref[...] = jnp.dot(h, ws_ref[...], preferred_element_type=jnp.float32)
    td_ref[...] = jnp.dot(h, wd_ref[...], preferred_element_type=jnp.float32)


def _edge_y(gs_ref, gd_ref, r_ref, we_ref, b_ref):
    rb = r_ref[...]
    d = jnp.sqrt(rb[:, 0:1] ** 2 + rb[:, 1:2] ** 2 + rb[:, 2:3] ** 2)
    centers = lax.broadcasted_iota(jnp.int32, (1, EF), 1).astype(
        jnp.float32) * (8.0 / (EF - 1))
    e = jnp.exp(-_GAMMA * (d - centers) ** 2)
    ep = jnp.dot(e, we_ref[...], preferred_element_type=jnp.float32)
    return gs_ref[...] + gd_ref[...] + ep + b_ref[...]


def _passA_kernel(gs_ref, gd_ref, r_ref, we_ref, b_ref, stats_ref):
    i = pl.program_id(0)

    @pl.when(i == 0)
    def _():
        stats_ref[...] = jnp.zeros_like(stats_ref)

    y = _edge_y(gs_ref, gd_ref, r_ref, we_ref, b_ref)
    rows = lax.broadcasted_iota(jnp.int32, (BE, 1), 0) + i * BE
    maskf = (rows < E).astype(jnp.float32)
    ym = y * maskf
    s = jnp.sum(ym, axis=0)
    s2 = jnp.sum(ym * y, axis=0)
    stats_ref[0, :] += s
    stats_ref[1, :] += s2


def _passB_kernel(gs_ref, gd_ref, r_ref, we_ref, b_ref, sc_ref, sh_ref,
                  msg_ref):
    y = _edge_y(gs_ref, gd_ref, r_ref, we_ref, b_ref)
    a = y * sc_ref[...] + sh_ref[...]
    inter = jax.nn.sigmoid(a[:, :NF])
    upd = jax.nn.softplus(a[:, NF:])
    msg_ref[...] = (inter * upd).astype(jnp.bfloat16)


def _nstats_kernel(agg_ref, stats_ref):
    i = pl.program_id(0)

    @pl.when(i == 0)
    def _():
        stats_ref[...] = jnp.zeros_like(stats_ref)

    rows = lax.broadcasted_iota(jnp.int32, (BN, 1), 0) + i * BN
    maskf = (rows < N).astype(jnp.float32)
    a = agg_ref[0]
    am = a * maskf
    stats_ref[0, :] += jnp.sum(am, axis=0)
    stats_ref[1, :] += jnp.sum(am * a, axis=0)


def _update_kernel(h_ref, agg_ref, sc_ref, sh_ref, hn_ref, hsum_ref):
    i = pl.program_id(0)

    @pl.when(i == 0)
    def _():
        hsum_ref[...] = jnp.zeros_like(hsum_ref)

    hn = jax.nn.softplus(h_ref[...] + agg_ref[0] * sc_ref[...] + sh_ref[...])
    hn_ref[...] = hn
    rows = lax.broadcasted_iota(jnp.int32, (BN, 1), 0) + i * BN
    maskf = (rows < N).astype(jnp.float32)
    hsum_ref[...] += jnp.sum(hn * maskf, axis=0, keepdims=True)


def _readout_kernel(hsum_ref, wfc_ref, bfc_ref, wout_ref, bout_ref, o_ref):
    feat = jax.nn.softplus(hsum_ref[...] / N)
    f2 = jax.nn.softplus(
        jnp.dot(feat, wfc_ref[...], preferred_element_type=jnp.float32)
        + bfc_ref[...]
    )
    o_ref[...] = jnp.sum(f2 * wout_ref[...], axis=1, keepdims=True) + bout_ref[...]


def _sc_mesh():
    return plsc.VectorSubcoreMesh(core_axis_name="c", subcore_axis_name="s")


@jax.jit
def _sc_gather(tsrc, tdst, src2d, dst2d):
    @functools.partial(
        pl.kernel,
        out_type=[
            jax.ShapeDtypeStruct((EP, 128), jnp.float32),
            jax.ShapeDtypeStruct((EP, 128), jnp.float32),
        ],
        mesh=_sc_mesh(),
    )
    def k(ts_hbm, td_hbm, si_hbm, di_hbm, gs_hbm, gd_hbm):
        def body(si, di, os, od):
            pltpu.sync_copy(ts_hbm.at[si.at[0]], os)
            pltpu.sync_copy(td_hbm.at[di.at[0]], od)

        pltpu.emit_pipeline(
            body,
            grid=(EP // GW,),
            in_specs=[
                pl.BlockSpec((1, GW), lambda i: (0, i)),
                pl.BlockSpec((1, GW), lambda i: (0, i)),
            ],
            out_specs=[
                pl.BlockSpec((GW, 128), lambda i: (i, 0)),
                pl.BlockSpec((GW, 128), lambda i: (i, 0)),
            ],
            core_axis_name=("c", "s"),
            dimension_semantics=(pltpu.PARALLEL,),
        )(si_hbm, di_hbm, gs_hbm, gd_hbm)

    return k(tsrc, tdst, src2d, dst2d)


def kernel(atom_features, r, edge_index, W_emb, b_emb, W_int, b_int, g_int,
           be_int, W_upd, b_upd, g_upd, be_upd, g_bn, be_bn, W_fc, b_fc,
           W_out, b_out):
    f32 = jnp.float32
    af = jnp.pad(atom_features, ((0, NP - N), (0, 0)))
    rp = jnp.pad(r, ((0, EP - E), (0, 0)))
    src = jnp.pad(edge_index[0], (0, EP - E)).reshape(1, EP)
    dst_flat = jnp.pad(edge_index[1], (0, EP - E), constant_values=SENTINEL)
    dst_g = jnp.pad(edge_index[1], (0, EP - E)).reshape(1, EP)

    wemb_t = W_emb.T
    bembr = b_emb.reshape(1, NF)
    ws_l = [jnp.concatenate([W_int[l][:, :NF].T, W_upd[l][:, :NF].T], axis=1)
            for l in range(L)]
    wd_l = [jnp.concatenate([W_int[l][:, NF:2 * NF].T,
                             W_upd[l][:, NF:2 * NF].T], axis=1)
            for l in range(L)]
    we_l = [jnp.concatenate([W_int[l][:, 2 * NF:].T,
                             W_upd[l][:, 2 * NF:].T], axis=1)
            for l in range(L)]
    b_l = [jnp.concatenate([b_int[l], b_upd[l]]).reshape(1, 2 * NF)
           for l in range(L)]
    g_l = [jnp.concatenate([g_int[l], g_upd[l]]) for l in range(L)]
    beta_l = [jnp.concatenate([be_int[l], be_upd[l]]) for l in range(L)]

    h = pl.pallas_call(
        _emb_kernel,
        grid=(NP // BN,),
        in_specs=[
            pl.BlockSpec((BN, 92), lambda i: (i, 0)),
            pl.BlockSpec((92, NF), lambda i: (0, 0)),
            pl.BlockSpec((1, NF), lambda i: (0, 0)),
        ],
        out_specs=pl.BlockSpec((BN, NF), lambda i: (i, 0)),
        out_shape=jax.ShapeDtypeStruct((NP, NF), f32),
    )(af, wemb_t, bembr)

    hsum = None
    for l in range(L):
        tsrc, tdst = pl.pallas_call(
            _proj_kernel,
            grid=(NP // BN,),
            in_specs=[
                pl.BlockSpec((BN, NF), lambda i: (i, 0)),
                pl.BlockSpec((NF, 2 * NF), lambda i: (0, 0)),
                pl.BlockSpec((NF, 2 * NF), lambda i: (0, 0)),
            ],
            out_specs=[
                pl.BlockSpec((BN, 2 * NF), lambda i: (i, 0)),
                pl.BlockSpec((BN, 2 * NF), lambda i: (i, 0)),
            ],
            out_shape=[
                jax.ShapeDtypeStruct((NP, 2 * NF), f32),
                jax.ShapeDtypeStruct((NP, 2 * NF), f32),
            ],
        )(h, ws_l[l], wd_l[l])

        gs, gd = _sc_gather(tsrc, tdst, src, dst_g)

        stats = pl.pallas_call(
            _passA_kernel,
            grid=(EP // BE,),
            in_specs=[
                pl.BlockSpec((BE, 2 * NF), lambda i: (i, 0)),
                pl.BlockSpec((BE, 2 * NF), lambda i: (i, 0)),
                pl.BlockSpec((BE, 3), lambda i: (i, 0)),
                pl.BlockSpec((EF, 2 * NF), lambda i: (0, 0)),
                pl.BlockSpec((1, 2 * NF), lambda i: (0, 0)),
            ],
            out_specs=pl.BlockSpec((2, 2 * NF), lambda i: (0, 0)),
            out_shape=jax.ShapeDtypeStruct((2, 2 * NF), f32),
        )(gs, gd, rp, we_l[l], b_l[l])
        mean = stats[0] / E
        var = stats[1] / E - mean * mean
        scale = g_l[l] / jnp.sqrt(var + 1e-5)
        shift = beta_l[l] - mean * scale

        msg = pl.pallas_call(
            _passB_kernel,
            grid=(EP // BE,),
            in_specs=[
                pl.BlockSpec((BE, 2 * NF), lambda i: (i, 0)),
                pl.BlockSpec((BE, 2 * NF), lambda i: (i, 0)),
                pl.BlockSpec((BE, 3), lambda i: (i, 0)),
                pl.BlockSpec((EF, 2 * NF), lambda i: (0, 0)),
                pl.BlockSpec((1, 2 * NF), lambda i: (0, 0)),
                pl.BlockSpec((1, 2 * NF), lambda i: (0, 0)),
                pl.BlockSpec((1, 2 * NF), lambda i: (0, 0)),
            ],
            out_specs=pl.BlockSpec((BE, NF), lambda i: (i, 0)),
            out_shape=jax.ShapeDtypeStruct((EP, NF), jnp.bfloat16),
        )(gs, gd, rp, we_l[l], b_l[l], scale.reshape(1, -1),
          shift.reshape(1, -1))

        half = jnp.where(dst_flat < NP, dst_flat // HALF, 0)
        row = jnp.where(dst_flat < NP, dst_flat % HALF, TRASH)
        flat = half * SP_ROWS + row
        agg = jax.ops.segment_sum(
            msg.astype(f32), flat,
            num_segments=2 * SP_ROWS).reshape(2, SP_ROWS, NF)

        nstats = pl.pallas_call(
            _nstats_kernel,
            grid=(NP // BN,),
            in_specs=[pl.BlockSpec((1, BN, NF),
                                   lambda i: (i // NHB, i % NHB, 0))],
            out_specs=pl.BlockSpec((2, NF), lambda i: (0, 0)),
            out_shape=jax.ShapeDtypeStruct((2, NF), f32),
        )(agg)
        nmean = nstats[0] / N
        nvar = nstats[1] / N - nmean * nmean
        nscale = g_bn[l] / jnp.sqrt(nvar + 1e-5)
        nshift = be_bn[l] - nmean * nscale

        h, hsum = pl.pallas_call(
            _update_kernel,
            grid=(NP // BN,),
            in_specs=[
                pl.BlockSpec((BN, NF), lambda i: (i, 0)),
                pl.BlockSpec((1, BN, NF), lambda i: (i // NHB, i % NHB, 0)),
                pl.BlockSpec((1, NF), lambda i: (0, 0)),
                pl.BlockSpec((1, NF), lambda i: (0, 0)),
            ],
            out_specs=[
                pl.BlockSpec((BN, NF), lambda i: (i, 0)),
                pl.BlockSpec((1, NF), lambda i: (0, 0)),
            ],
            out_shape=[
                jax.ShapeDtypeStruct((NP, NF), f32),
                jax.ShapeDtypeStruct((1, NF), f32),
            ],
        )(h, agg, nscale.reshape(1, -1), nshift.reshape(1, -1))

    out = pl.pallas_call(
        _readout_kernel,
        out_shape=jax.ShapeDtypeStruct((1, 1), f32),
    )(hsum, W_fc.T, b_fc.reshape(1, -1), W_out, b_out.reshape(1, -1))
    return jnp.squeeze(out)


# speedup vs baseline: 1.0925x; 1.0925x over previous
import functools

import jax
import jax.numpy as jnp
from jax import lax
from jax.experimental import pallas as pl
from jax.experimental.pallas import tpu as pltpu
from jax.experimental.pallas import tpu_sc as plsc

N = 50000
E = 800000
EF = 40
L = 3
NF = 64

NP = 51200
HALF = 25600
NHB = 25
EP = 802816
GW = 128
BE = 4096
BN = 1024
SP_ROWS = 25664
TRASH = 25600
SENTINEL = 60000

_GAMMA = (EF - 1) / 8.0


def _emb_kernel(af_ref, w_ref, b_ref, h_ref):
    h_ref[...] = (
        jnp.dot(af_ref[...], w_ref[...], preferred_element_type=jnp.float32)
        + b_ref[...]
    )


def _proj_kernel(h_ref, ws_ref, wd_ref, ts_ref, td_ref):
    h = h_ref[...]
    ts_ref[...] = jnp.dot(h, ws_ref[...], preferred_element_type=jnp.float32)
    td_ref[...] = jnp.dot(h, wd_ref[...], preferred_element_type=jnp.float32)


def _edge_y(gs_ref, gd_ref, r_ref, we_ref, b_ref):
    rb = r_ref[...]
    d = jnp.sqrt(rb[:, 0:1] ** 2 + rb[:, 1:2] ** 2 + rb[:, 2:3] ** 2)
    centers = lax.broadcasted_iota(jnp.int32, (1, EF), 1).astype(
        jnp.float32) * (8.0 / (EF - 1))
    e = jnp.exp(-_GAMMA * (d - centers) ** 2)
    ep = jnp.dot(e, we_ref[...], preferred_element_type=jnp.float32)
    return gs_ref[...] + gd_ref[...] + ep + b_ref[...]


def _passA_kernel(gs_ref, gd_ref, r_ref, we_ref, b_ref, stats_ref):
    i = pl.program_id(0)

    @pl.when(i == 0)
    def _():
        stats_ref[...] = jnp.zeros_like(stats_ref)

    y = _edge_y(gs_ref, gd_ref, r_ref, we_ref, b_ref)
    rows = lax.broadcasted_iota(jnp.int32, (BE, 1), 0) + i * BE
    maskf = (rows < E).astype(jnp.float32)
    ym = y * maskf
    s = jnp.sum(ym, axis=0)
    s2 = jnp.sum(ym * y, axis=0)
    stats_ref[0, :] += s
    stats_ref[1, :] += s2


def _passB_kernel(gs_ref, gd_ref, r_ref, we_ref, b_ref, sc_ref, sh_ref,
                  msg_ref):
    y = _edge_y(gs_ref, gd_ref, r_ref, we_ref, b_ref)
    a = y * sc_ref[...] + sh_ref[...]
    inter = jax.nn.sigmoid(a[:, :NF])
    upd = jax.nn.softplus(a[:, NF:])
    msg_ref[...] = (inter * upd).astype(jnp.bfloat16)


def _nstats_kernel(agg_ref, stats_ref):
    i = pl.program_id(0)

    @pl.when(i == 0)
    def _():
        stats_ref[...] = jnp.zeros_like(stats_ref)

    rows = lax.broadcasted_iota(jnp.int32, (BN, 1), 0) + i * BN
    maskf = (rows < N).astype(jnp.float32)
    a = agg_ref[0]
    am = a * maskf
    stats_ref[0, :] += jnp.sum(am, axis=0)
    stats_ref[1, :] += jnp.sum(am * a, axis=0)


def _update_kernel(h_ref, agg_ref, sc_ref, sh_ref, hn_ref, hsum_ref):
    i = pl.program_id(0)

    @pl.when(i == 0)
    def _():
        hsum_ref[...] = jnp.zeros_like(hsum_ref)

    hn = jax.nn.softplus(h_ref[...] + agg_ref[0] * sc_ref[...] + sh_ref[...])
    hn_ref[...] = hn
    rows = lax.broadcasted_iota(jnp.int32, (BN, 1), 0) + i * BN
    maskf = (rows < N).astype(jnp.float32)
    hsum_ref[...] += jnp.sum(hn * maskf, axis=0, keepdims=True)


def _readout_kernel(hsum_ref, wfc_ref, bfc_ref, wout_ref, bout_ref, o_ref):
    feat = jax.nn.softplus(hsum_ref[...] / N)
    f2 = jax.nn.softplus(
        jnp.dot(feat, wfc_ref[...], preferred_element_type=jnp.float32)
        + bfc_ref[...]
    )
    o_ref[...] = jnp.sum(f2 * wout_ref[...], axis=1, keepdims=True) + bout_ref[...]


def _sc_mesh():
    return plsc.VectorSubcoreMesh(core_axis_name="c", subcore_axis_name="s")


@jax.jit
def _sc_gather(tsrc, tdst, src2d, dst2d):
    @functools.partial(
        pl.kernel,
        out_type=[
            jax.ShapeDtypeStruct((EP, 128), jnp.float32),
            jax.ShapeDtypeStruct((EP, 128), jnp.float32),
        ],
        mesh=_sc_mesh(),
    )
    def k(ts_hbm, td_hbm, si_hbm, di_hbm, gs_hbm, gd_hbm):
        def body(si, di, os, od):
            pltpu.sync_copy(ts_hbm.at[si.at[0]], os)
            pltpu.sync_copy(td_hbm.at[di.at[0]], od)

        pltpu.emit_pipeline(
            body,
            grid=(EP // GW,),
            in_specs=[
                pl.BlockSpec((1, GW), lambda i: (0, i)),
                pl.BlockSpec((1, GW), lambda i: (0, i)),
            ],
            out_specs=[
                pl.BlockSpec((GW, 128), lambda i: (i, 0)),
                pl.BlockSpec((GW, 128), lambda i: (i, 0)),
            ],
            core_axis_name=("c", "s"),
            dimension_semantics=(pltpu.PARALLEL,),
        )(si_hbm, di_hbm, gs_hbm, gd_hbm)

    return k(tsrc, tdst, src2d, dst2d)


def kernel(atom_features, r, edge_index, W_emb, b_emb, W_int, b_int, g_int,
           be_int, W_upd, b_upd, g_upd, be_upd, g_bn, be_bn, W_fc, b_fc,
           W_out, b_out):
    f32 = jnp.float32
    af = jnp.pad(atom_features, ((0, NP - N), (0, 0)))
    rp = jnp.pad(r, ((0, EP - E), (0, 0)))
    src = jnp.pad(edge_index[0], (0, EP - E)).reshape(1, EP)
    dst_flat = jnp.pad(edge_index[1], (0, EP - E), constant_values=SENTINEL)
    dst_g = jnp.pad(edge_index[1], (0, EP - E)).reshape(1, EP)

    wemb_t = W_emb.T
    bembr = b_emb.reshape(1, NF)
    ws_l = [jnp.concatenate([W_int[l][:, :NF].T, W_upd[l][:, :NF].T], axis=1)
            for l in range(L)]
    wd_l = [jnp.concatenate([W_int[l][:, NF:2 * NF].T,
                             W_upd[l][:, NF:2 * NF].T], axis=1)
            for l in range(L)]
    we_l = [jnp.concatenate([W_int[l][:, 2 * NF:].T,
                             W_upd[l][:, 2 * NF:].T], axis=1)
            for l in range(L)]
    b_l = [jnp.concatenate([b_int[l], b_upd[l]]).reshape(1, 2 * NF)
           for l in range(L)]
    g_l = [jnp.concatenate([g_int[l], g_upd[l]]) for l in range(L)]
    beta_l = [jnp.concatenate([be_int[l], be_upd[l]]) for l in range(L)]

    h = pl.pallas_call(
        _emb_kernel,
        grid=(NP // BN,),
        in_specs=[
            pl.BlockSpec((BN, 92), lambda i: (i, 0)),
            pl.BlockSpec((92, NF), lambda i: (0, 0)),
            pl.BlockSpec((1, NF), lambda i: (0, 0)),
        ],
        out_specs=pl.BlockSpec((BN, NF), lambda i: (i, 0)),
        out_shape=jax.ShapeDtypeStruct((NP, NF), f32),
    )(af, wemb_t, bembr)

    hsum = None
    for l in range(L):
        tsrc, tdst = pl.pallas_call(
            _proj_kernel,
            grid=(NP // BN,),
            in_specs=[
                pl.BlockSpec((BN, NF), lambda i: (i, 0)),
                pl.BlockSpec((NF, 2 * NF), lambda i: (0, 0)),
                pl.BlockSpec((NF, 2 * NF), lambda i: (0, 0)),
            ],
            out_specs=[
                pl.BlockSpec((BN, 2 * NF), lambda i: (i, 0)),
                pl.BlockSpec((BN, 2 * NF), lambda i: (i, 0)),
            ],
            out_shape=[
                jax.ShapeDtypeStruct((NP, 2 * NF), f32),
                jax.ShapeDtypeStruct((NP, 2 * NF), f32),
            ],
        )(h, ws_l[l], wd_l[l])

        gs, gd = _sc_gather(tsrc, tdst, src, dst_g)

        stats = pl.pallas_call(
            _passA_kernel,
            grid=(EP // BE,),
            in_specs=[
                pl.BlockSpec((BE, 2 * NF), lambda i: (i, 0)),
                pl.BlockSpec((BE, 2 * NF), lambda i: (i, 0)),
                pl.BlockSpec((BE, 3), lambda i: (i, 0)),
                pl.BlockSpec((EF, 2 * NF), lambda i: (0, 0)),
                pl.BlockSpec((1, 2 * NF), lambda i: (0, 0)),
            ],
            out_specs=pl.BlockSpec((2, 2 * NF), lambda i: (0, 0)),
            out_shape=jax.ShapeDtypeStruct((2, 2 * NF), f32),
        )(gs, gd, rp, we_l[l], b_l[l])
        mean = stats[0] / E
        var = stats[1] / E - mean * mean
        scale = g_l[l] / jnp.sqrt(var + 1e-5)
        shift = beta_l[l] - mean * scale

        msg = pl.pallas_call(
            _passB_kernel,
            grid=(EP // BE,),
            in_specs=[
                pl.BlockSpec((BE, 2 * NF), lambda i: (i, 0)),
                pl.BlockSpec((BE, 2 * NF), lambda i: (i, 0)),
                pl.BlockSpec((BE, 3), lambda i: (i, 0)),
                pl.BlockSpec((EF, 2 * NF), lambda i: (0, 0)),
                pl.BlockSpec((1, 2 * NF), lambda i: (0, 0)),
                pl.BlockSpec((1, 2 * NF), lambda i: (0, 0)),
                pl.BlockSpec((1, 2 * NF), lambda i: (0, 0)),
            ],
            out_specs=pl.BlockSpec((BE, NF), lambda i: (i, 0)),
            out_shape=jax.ShapeDtypeStruct((EP, NF), jnp.bfloat16),
        )(gs, gd, rp, we_l[l], b_l[l], scale.reshape(1, -1),
          shift.reshape(1, -1))

        half = jnp.where(dst_flat < NP, dst_flat // HALF, 0)
        row = jnp.where(dst_flat < NP, dst_flat % HALF, TRASH)
        flat = half * SP_ROWS + row
        agg = jax.ops.segment_sum(
            msg, flat, num_segments=2 * SP_ROWS,
        ).astype(f32).reshape(2, SP_ROWS, NF)

        nstats = pl.pallas_call(
            _nstats_kernel,
            grid=(NP // BN,),
            in_specs=[pl.BlockSpec((1, BN, NF),
                                   lambda i: (i // NHB, i % NHB, 0))],
            out_specs=pl.BlockSpec((2, NF), lambda i: (0, 0)),
            out_shape=jax.ShapeDtypeStruct((2, NF), f32),
        )(agg)
        nmean = nstats[0] / N
        nvar = nstats[1] / N - nmean * nmean
        nscale = g_bn[l] / jnp.sqrt(nvar + 1e-5)
        nshift = be_bn[l] - nmean * nscale

        h, hsum = pl.pallas_call(
            _update_kernel,
            grid=(NP // BN,),
            in_specs=[
                pl.BlockSpec((BN, NF), lambda i: (i, 0)),
                pl.BlockSpec((1, BN, NF), lambda i: (i // NHB, i % NHB, 0)),
                pl.BlockSpec((1, NF), lambda i: (0, 0)),
                pl.BlockSpec((1, NF), lambda i: (0, 0)),
            ],
            out_specs=[
                pl.BlockSpec((BN, NF), lambda i: (i, 0)),
                pl.BlockSpec((1, NF), lambda i: (0, 0)),
            ],
            out_shape=[
                jax.ShapeDtypeStruct((NP, NF), f32),
                jax.ShapeDtypeStruct((1, NF), f32),
            ],
        )(h, agg, nscale.reshape(1, -1), nshift.reshape(1, -1))

    out = pl.pallas_call(
        _readout_kernel,
        out_shape=jax.ShapeDtypeStruct((1, 1), f32),
    )(hsum, W_fc.T, b_fc.reshape(1, -1), W_out, b_out.reshape(1, -1))
    return jnp.squeeze(out)
